# baseline (speedup 1.0000x reference)
import math

import jax
import jax.numpy as jnp
from jax import lax
from jax.experimental import pallas as pl
from jax.experimental.pallas import tpu as pltpu

_END = 2
_LANES = 128
_ITEMS = 8


def _body(cur_ref, pcol_ref, ecol_ref, erow_ref, state_ref, dec_ref,
          outp_ref, outv_ref, oute_ref, outd_ref, outs_ref,
          m_ref, fi_ref):
    I, beam, V = cur_ref.shape
    nch = V // _LANES
    L = _LANES
    rows = I * beam
    x = cur_ref[...]
    pcol = pcol_ref[...]
    ecol = ecol_ref[...]
    erow = erow_ref[...]

    neg = jnp.float32(-jnp.inf)
    BIG = jnp.int32(1 << 30)

    s = jnp.sum(jnp.exp(x), axis=-1, keepdims=True)
    c = pcol - jnp.log(s)

    x3 = x.reshape(rows, nch, L)
    ch = lax.broadcasted_iota(jnp.int32, (rows, nch, L), 1)

    m1 = jnp.max(x3, axis=1)
    a1 = jnp.min(jnp.where(x3 == m1[:, None, :], ch, BIG), axis=1)
    m2 = jnp.max(jnp.where(ch == a1[:, None, :], neg, x3), axis=1)
    a2 = jnp.min(jnp.where((x3 == m2[:, None, :]) & (ch != a1[:, None, :]),
                           ch, BIG), axis=1)

    cr = c.reshape(rows, 1)
    pr = pcol.reshape(rows, 1)
    er = ecol.reshape(rows, 1) > 0
    lane = lax.broadcasted_iota(jnp.int32, (rows, L), 1)
    bsub = lax.rem(lax.broadcasted_iota(jnp.int32, (rows, L), 0),
                   jnp.int32(beam))
    base = bsub * V + lane

    h1 = m1 + cr
    f1 = base + a1 * L
    h2 = m2 + cr
    f2 = base + a2 * L

    e_lane = lane == (_END % L)
    e_flat = bsub * V + _END
    h1 = jnp.where(er, jnp.where(e_lane, pr, neg), h1)
    f1 = jnp.where(er, jnp.where(e_lane, e_flat, BIG), f1)
    h2 = jnp.where(er, neg, h2)
    f2 = jnp.where(er, BIG, f2)

    h1 = h1.reshape(I, beam, L)
    f1 = f1.reshape(I, beam, L)
    h2 = h2.reshape(I, beam, L)
    f2 = f2.reshape(I, beam, L)

    t_cnt = jnp.zeros((I, beam, L), jnp.int32)
    danger = jnp.full((I, 1, 1), neg)
    deg = jnp.zeros((I, 1, 1), jnp.bool_)

    l8 = lax.broadcasted_iota(jnp.int32, (I, 1, beam), 2)
    sub8 = lax.broadcasted_iota(jnp.int32, (I, beam, beam), 1)
    lan8 = lax.broadcasted_iota(jnp.int32, (I, beam, beam), 2)
    val_row = jnp.zeros((I, 1, beam), jnp.float32)
    voc_row = jnp.zeros((I, 1, beam), jnp.int32)
    W = jnp.zeros((I, beam, beam), jnp.float32)
    vlog2 = int(math.log2(V))
    removed = []

    for k in range(beam):
        m_fast = jnp.max(h1, axis=(1, 2), keepdims=True)
        fi_fast = jnp.min(jnp.where(h1 == m_fast, f1, BIG),
                          axis=(1, 2), keepdims=True)
        safe = jnp.logical_and(jnp.logical_not(deg), m_fast > danger)
        all_safe = jnp.all(safe)

        m_ref[...] = jnp.broadcast_to(m_fast, (I, 1, L))
        fi_ref[...] = jnp.broadcast_to(fi_fast, (I, 1, L))

        @pl.when(jnp.logical_not(all_safe))
        def _(removed=tuple(removed)):
            x4 = x.reshape(I, beam, nch, L)
            f4 = (lax.broadcasted_iota(jnp.int32, (I, beam, nch, L), 1) * V
                  + lax.broadcasted_iota(jnp.int32, (I, beam, nch, L), 2) * L
                  + lax.broadcasted_iota(jnp.int32, (I, beam, nch, L), 3))
            t4 = x4 + c[:, :, :, None]
            e4 = (ecol > 0)[:, :, :, None]
            bflat = (lax.broadcasted_iota(jnp.int32, (I, beam, 1, 1), 1) * V
                     + _END)
            t4 = jnp.where(e4, jnp.where(f4 == bflat, pcol[:, :, :, None],
                                         neg), t4)
            ts = t4.reshape(I, beam * nch, L)
            fs = f4.reshape(I, beam * nch, L)
            rm = jnp.zeros((I, beam * nch, L), jnp.bool_)
            for r in removed:
                rm = jnp.logical_or(rm, fs == r)
            ts = jnp.where(rm, neg, ts)
            m_slow = jnp.max(ts, axis=(1, 2), keepdims=True)
            fi_slow = jnp.min(jnp.where(ts == m_slow, fs, BIG),
                              axis=(1, 2), keepdims=True)
            m_ref[...] = jnp.broadcast_to(m_slow, (I, 1, L))
            fi_ref[...] = jnp.broadcast_to(fi_slow, (I, 1, L))

        m = m_ref[:, :, 0:1]
        fi = fi_ref[:, :, 0:1]
        removed.append(fi)

        colm = f1 == fi
        second_pop = jnp.any(jnp.logical_and(colm, t_cnt == 1),
                             axis=(1, 2), keepdims=True)
        danger = jnp.where(jnp.logical_and(safe, second_pop),
                           jnp.maximum(danger, m), danger)
        t_cnt = t_cnt + colm.astype(jnp.int32)
        h1 = jnp.where(colm, h2, h1)
        f1 = jnp.where(colm, f2, f1)
        h2 = jnp.where(colm, neg, h2)
        f2 = jnp.where(colm, BIG, f2)
        deg = jnp.logical_or(deg, jnp.logical_not(safe))

        vk = fi & (V - 1) if (1 << vlog2) == V else fi % V
        bk = lax.shift_right_logical(fi, vlog2) if (1 << vlog2) == V else fi // V
        val_row = jnp.where(l8 == k, m, val_row)
        voc_row = jnp.where(l8 == k, vk, voc_row)
        W = W + jnp.where((sub8 == k) & (lan8 == bk), 1.0, 0.0)

    outp_ref[...] = val_row
    outv_ref[...] = voc_row
    oute_ref[...] = ((erow > 0) | (voc_row == _END)).astype(jnp.int32)
    for i in range(I):
        dec_i = dec_ref[i].astype(jnp.float32)
        outd_ref[i] = lax.dot_general(
            dec_i, W[i], (((1,), (1,)), ((), ())),
            preferred_element_type=jnp.float32).astype(jnp.int32)
        outs_ref[i] = lax.dot_general(
            W[i], state_ref[i], (((1,), (0,)), ((), ())),
            preferred_element_type=jnp.float32)


def kernel(cur_proba, proba, is_ended, state, decodeds):
    batch, beam, V = cur_proba.shape
    d = state.shape[-1]
    t = decodeds.shape[0]
    I = _ITEMS
    pcol = proba.reshape(batch, beam, 1)
    ecol = is_ended.astype(jnp.int32).reshape(batch, beam, 1)
    erow = is_ended.astype(jnp.int32).reshape(batch, 1, beam)
    dec3 = decodeds.astype(jnp.int32).transpose(1, 0, 2)

    outs = pl.pallas_call(
        _body,
        grid=(batch // I,),
        in_specs=[
            pl.BlockSpec((I, beam, V), lambda b: (b, 0, 0)),
            pl.BlockSpec((I, beam, 1), lambda b: (b, 0, 0)),
            pl.BlockSpec((I, beam, 1), lambda b: (b, 0, 0)),
            pl.BlockSpec((I, 1, beam), lambda b: (b, 0, 0)),
            pl.BlockSpec((I, beam, d), lambda b: (b, 0, 0)),
            pl.BlockSpec((I, t, beam), lambda b: (b, 0, 0)),
        ],
        out_specs=[
            pl.BlockSpec((I, 1, beam), lambda b: (b, 0, 0)),
            pl.BlockSpec((I, 1, beam), lambda b: (b, 0, 0)),
            pl.BlockSpec((I, 1, beam), lambda b: (b, 0, 0)),
            pl.BlockSpec((I, t, beam), lambda b: (b, 0, 0)),
            pl.BlockSpec((I, beam, d), lambda b: (b, 0, 0)),
        ],
        out_shape=[
            jax.ShapeDtypeStruct((batch, 1, beam), jnp.float32),
            jax.ShapeDtypeStruct((batch, 1, beam), jnp.int32),
            jax.ShapeDtypeStruct((batch, 1, beam), jnp.int32),
            jax.ShapeDtypeStruct((batch, t, beam), jnp.int32),
            jax.ShapeDtypeStruct((batch, beam, d), jnp.float32),
        ],
        scratch_shapes=[
            pltpu.VMEM((I, 1, _LANES), jnp.float32),
            pltpu.VMEM((I, 1, _LANES), jnp.int32),
        ],
    )(cur_proba, pcol, ecol, erow, state, dec3)

    new_proba3, voc3, ended3, decg3, new_state = outs
    new_proba = new_proba3.reshape(batch, beam)
    topk_voc = voc3.reshape(batch, beam)
    new_is_ended = ended3.reshape(batch, beam).astype(bool)
    gathered_dec = decg3.transpose(1, 0, 2)
    new_decodeds = jnp.concatenate([gathered_dec, topk_voc[None]], axis=0)
    cur_input = topk_voc.reshape(-1)
    return new_proba, new_decodeds, new_is_ended, new_state, cur_input


# speedup vs baseline: 1.5139x; 1.5139x over previous
import math

import jax
import jax.numpy as jnp
from jax import lax
from jax.experimental import pallas as pl

_END = 2
_LANES = 128


def _body(cur_ref, pcol_ref, ecol_ref, erow_ref, state_ref, dec_ref,
          outp_ref, outv_ref, oute_ref, outd_ref, outs_ref):
    beam, V = cur_ref.shape[1], cur_ref.shape[2]
    nch = V // _LANES
    L = _LANES
    x = cur_ref[0]
    pcol = pcol_ref[0]
    ecol = ecol_ref[0]
    erow = erow_ref[0]

    neg = jnp.float32(-jnp.inf)
    BIG = jnp.int32(1 << 30)
    vlog2 = int(math.log2(V))

    s = jnp.sum(jnp.exp(x), axis=-1, keepdims=True)
    c = pcol - jnp.log(s)

    x3 = x.reshape(beam, nch, L)
    ch = lax.broadcasted_iota(jnp.int32, (beam, nch, L), 1)

    m1 = jnp.max(x3, axis=1)
    a1 = jnp.min(jnp.where(x3 == m1[:, None, :], ch, BIG), axis=1)
    m2 = jnp.max(jnp.where(ch == a1[:, None, :], neg, x3), axis=1)
    a2 = jnp.min(jnp.where((x3 == m2[:, None, :]) & (ch != a1[:, None, :]),
                           ch, BIG), axis=1)

    bsub = lax.broadcasted_iota(jnp.int32, (beam, L), 0)
    lane = lax.broadcasted_iota(jnp.int32, (beam, L), 1)
    base = bsub * V + lane

    endm = ecol > 0
    e_lane = lane == (_END % L)
    e_flat = bsub * V + _END

    h1 = jnp.where(endm, jnp.where(e_lane, pcol, neg), m1 + c)
    f1 = jnp.where(endm, jnp.where(e_lane, e_flat, BIG), base + a1 * L)
    h2 = jnp.where(endm, neg, m2 + c)
    f2 = jnp.where(endm, BIG, base + a2 * L)

    t_one = jnp.zeros((beam, L), jnp.bool_)
    danger = neg
    unsafe = jnp.bool_(False)

    l8 = lax.broadcasted_iota(jnp.int32, (1, beam), 1)
    sub8 = lax.broadcasted_iota(jnp.int32, (beam, beam), 0)
    lan8 = lax.broadcasted_iota(jnp.int32, (beam, beam), 1)
    val_row = jnp.zeros((1, beam), jnp.float32)
    voc_row = jnp.zeros((1, beam), jnp.int32)
    W = jnp.zeros((beam, beam), jnp.float32)

    for k in range(beam):
        m = jnp.max(h1)
        fi = jnp.min(jnp.where(h1 == m, f1, BIG))
        unsafe = jnp.logical_or(unsafe, jnp.logical_not(m > danger))
        colm = f1 == fi
        second_pop = jnp.any(jnp.logical_and(colm, t_one))
        danger = jnp.where(second_pop, jnp.maximum(danger, m), danger)
        t_one = jnp.logical_or(t_one, colm)
        h1 = jnp.where(colm, h2, h1)
        f1 = jnp.where(colm, f2, f1)
        h2 = jnp.where(colm, neg, h2)
        f2 = jnp.where(colm, BIG, f2)

        vk = fi & (V - 1)
        bk = lax.shift_right_logical(fi, vlog2)
        val_row = jnp.where(l8 == k, m, val_row)
        voc_row = jnp.where(l8 == k, vk, voc_row)
        W = W + jnp.where((sub8 == k) & (lan8 == bk), 1.0, 0.0)

    outp_ref[0] = val_row
    outv_ref[0] = voc_row
    oute_ref[0] = ((erow > 0) | (voc_row == _END)).astype(jnp.int32)
    dec = dec_ref[0].astype(jnp.float32)
    outd_ref[0] = lax.dot_general(
        dec, W, (((1,), (1,)), ((), ())),
        preferred_element_type=jnp.float32).astype(jnp.int32)
    outs_ref[0] = lax.dot_general(
        W, state_ref[0], (((1,), (0,)), ((), ())),
        preferred_element_type=jnp.float32)

    @pl.when(unsafe)
    def _():
        col = lax.broadcasted_iota(jnp.int32, (beam, V), 1)
        bs2 = lax.broadcasted_iota(jnp.int32, (beam, V), 0)
        flat = bs2 * V + col
        total = jnp.where(endm, jnp.where(col == _END, pcol, neg), x + c)

        vr = jnp.zeros((1, beam), jnp.float32)
        vo = jnp.zeros((1, beam), jnp.int32)
        Ws = jnp.zeros((beam, beam), jnp.float32)
        tt = total
        for k in range(beam):
            mm = jnp.max(tt)
            ff = jnp.min(jnp.where(tt == mm, flat, BIG))
            tt = jnp.where(flat == ff, neg, tt)
            vv = ff & (V - 1)
            bb = lax.shift_right_logical(ff, vlog2)
            vr = jnp.where(l8 == k, mm, vr)
            vo = jnp.where(l8 == k, vv, vo)
            Ws = Ws + jnp.where((sub8 == k) & (lan8 == bb), 1.0, 0.0)

        outp_ref[0] = vr
        outv_ref[0] = vo
        oute_ref[0] = ((erow > 0) | (vo == _END)).astype(jnp.int32)
        dec2 = dec_ref[0].astype(jnp.float32)
        outd_ref[0] = lax.dot_general(
            dec2, Ws, (((1,), (1,)), ((), ())),
            preferred_element_type=jnp.float32).astype(jnp.int32)
        outs_ref[0] = lax.dot_general(
            Ws, state_ref[0], (((1,), (0,)), ((), ())),
            preferred_element_type=jnp.float32)


def kernel(cur_proba, proba, is_ended, state, decodeds):
    batch, beam, V = cur_proba.shape
    d = state.shape[-1]
    t = decodeds.shape[0]
    pcol = proba.reshape(batch, beam, 1)
    ecol = is_ended.astype(jnp.int32).reshape(batch, beam, 1)
    erow = is_ended.astype(jnp.int32).reshape(batch, 1, beam)
    dec3 = decodeds.astype(jnp.int32).transpose(1, 0, 2)

    outs = pl.pallas_call(
        _body,
        grid=(batch,),
        in_specs=[
            pl.BlockSpec((1, beam, V), lambda b: (b, 0, 0)),
            pl.BlockSpec((1, beam, 1), lambda b: (b, 0, 0)),
            pl.BlockSpec((1, beam, 1), lambda b: (b, 0, 0)),
            pl.BlockSpec((1, 1, beam), lambda b: (b, 0, 0)),
            pl.BlockSpec((1, beam, d), lambda b: (b, 0, 0)),
            pl.BlockSpec((1, t, beam), lambda b: (b, 0, 0)),
        ],
        out_specs=[
            pl.BlockSpec((1, 1, beam), lambda b: (b, 0, 0)),
            pl.BlockSpec((1, 1, beam), lambda b: (b, 0, 0)),
            pl.BlockSpec((1, 1, beam), lambda b: (b, 0, 0)),
            pl.BlockSpec((1, t, beam), lambda b: (b, 0, 0)),
            pl.BlockSpec((1, beam, d), lambda b: (b, 0, 0)),
        ],
        out_shape=[
            jax.ShapeDtypeStruct((batch, 1, beam), jnp.float32),
            jax.ShapeDtypeStruct((batch, 1, beam), jnp.int32),
            jax.ShapeDtypeStruct((batch, 1, beam), jnp.int32),
            jax.ShapeDtypeStruct((batch, t, beam), jnp.int32),
            jax.ShapeDtypeStruct((batch, beam, d), jnp.float32),
        ],
    )(cur_proba, pcol, ecol, erow, state, dec3)

    new_proba3, voc3, ended3, decg3, new_state = outs
    new_proba = new_proba3.reshape(batch, beam)
    topk_voc = voc3.reshape(batch, beam)
    new_is_ended = ended3.reshape(batch, beam).astype(bool)
    gathered_dec = decg3.transpose(1, 0, 2)
    new_decodeds = jnp.concatenate([gathered_dec, topk_voc[None]], axis=0)
    cur_input = topk_voc.reshape(-1)
    return new_proba, new_decodeds, new_is_ended, new_state, cur_input
